# Initial kernel scaffold; baseline (speedup 1.0000x reference)
import functools

import jax
import jax.numpy as jnp
from jax import lax
from jax.experimental import pallas as pl
from jax.experimental.pallas import tpu as pltpu
from jax.experimental.pallas import tpu_sc as plsc

NC = 2
NS = 16
N_PAD = 10240
ROWS_PER_TILE = N_PAD // NS
E_PAD = 163840
CHUNK = 128
RB = 256
GRID = N_PAD // RB
DH = 256
DHH = DH // NC
DO_PAD = 16


def _sc_mesh():
    return plsc.VectorSubcoreMesh(core_axis_name="c", subcore_axis_name="s")


def _deg_body(dst_hbm, ones_hbm, zeros_hbm, out_hbm, dst_v, ones_v, acc):
    cid = lax.axis_index("c")
    sid = lax.axis_index("s")
    r0 = sid * ROWS_PER_TILE
    pltpu.sync_copy(zeros_hbm, acc.at[pl.ds(r0, ROWS_PER_TILE)])
    pltpu.sync_copy(ones_hbm, ones_v)
    plsc.subcore_barrier()
    e_per_tile = E_PAD // (NC * NS)
    base = cid * (E_PAD // NC) + sid * e_per_tile

    def body(k, carry):
        pltpu.sync_copy(dst_hbm.at[pl.ds(base + k * CHUNK, CHUNK)], dst_v)
        pltpu.sync_copy(ones_v, acc.at[dst_v], add=True)
        return carry

    lax.fori_loop(0, e_per_tile // CHUNK, body, 0)
    plsc.subcore_barrier()
    pltpu.sync_copy(acc.at[pl.ds(r0, ROWS_PER_TILE)],
                    out_hbm.at[cid, pl.ds(r0, ROWS_PER_TILE)])


_deg_call = functools.partial(
    pl.kernel,
    out_type=jax.ShapeDtypeStruct((NC, N_PAD, DO_PAD), jnp.float32),
    mesh=_sc_mesh(),
    scratch_types=[
        pltpu.VMEM((CHUNK,), jnp.int32),
        pltpu.VMEM((CHUNK, DO_PAD), jnp.float32),
        pltpu.VMEM_SHARED((N_PAD, DO_PAD), jnp.float32),
    ],
)(_deg_body)


def _agg1_body(hs_hbm, srcs_hbm, dst_hbm, zeros_hbm, out_hbm,
               src_v, dst_v, rows_v, acc, sem):
    cid = lax.axis_index("c")
    sid = lax.axis_index("s")
    r0 = sid * ROWS_PER_TILE
    pltpu.sync_copy(zeros_hbm, acc.at[pl.ds(r0, ROWS_PER_TILE)])
    plsc.subcore_barrier()
    e_per_tile = E_PAD // NS
    base = sid * e_per_tile

    def body(k, carry):
        b = base + k * CHUNK
        pltpu.sync_copy(srcs_hbm.at[cid, pl.ds(b, CHUNK)], src_v)
        pltpu.sync_copy(dst_hbm.at[pl.ds(b, CHUNK)], dst_v)
        pltpu.async_copy(hs_hbm.at[src_v], rows_v, sem).wait()
        pltpu.sync_copy(rows_v, acc.at[dst_v], add=True)
        return carry

    lax.fori_loop(0, e_per_tile // CHUNK, body, 0)
    plsc.subcore_barrier()
    pltpu.sync_copy(acc.at[pl.ds(r0, ROWS_PER_TILE)],
                    out_hbm.at[cid, pl.ds(r0, ROWS_PER_TILE)])


_agg1_call = functools.partial(
    pl.kernel,
    out_type=jax.ShapeDtypeStruct((NC, N_PAD, DHH), jnp.float32),
    mesh=_sc_mesh(),
    scratch_types=[
        pltpu.VMEM((CHUNK,), jnp.int32),
        pltpu.VMEM((CHUNK,), jnp.int32),
        pltpu.VMEM((CHUNK, DHH), jnp.float32),
        pltpu.VMEM_SHARED((N_PAD, DHH), jnp.float32),
        pltpu.SemaphoreType.DMA,
    ],
)(_agg1_body)


def _agg2_body(zs_hbm, src_hbm, dst_hbm, zeros_hbm, out_hbm,
               src_v, dst_v, rows_v, acc, sem):
    cid = lax.axis_index("c")
    sid = lax.axis_index("s")
    r0 = sid * ROWS_PER_TILE
    pltpu.sync_copy(zeros_hbm, acc.at[pl.ds(r0, ROWS_PER_TILE)])
    plsc.subcore_barrier()
    e_per_tile = E_PAD // (NC * NS)
    base = cid * (E_PAD // NC) + sid * e_per_tile

    def body(k, carry):
        b = base + k * CHUNK
        pltpu.sync_copy(src_hbm.at[pl.ds(b, CHUNK)], src_v)
        pltpu.sync_copy(dst_hbm.at[pl.ds(b, CHUNK)], dst_v)
        pltpu.async_copy(zs_hbm.at[src_v], rows_v, sem).wait()
        pltpu.sync_copy(rows_v, acc.at[dst_v], add=True)
        return carry

    lax.fori_loop(0, e_per_tile // CHUNK, body, 0)
    plsc.subcore_barrier()
    pltpu.sync_copy(acc.at[pl.ds(r0, ROWS_PER_TILE)],
                    out_hbm.at[cid, pl.ds(r0, ROWS_PER_TILE)])


_agg2_call = functools.partial(
    pl.kernel,
    out_type=jax.ShapeDtypeStruct((NC, N_PAD, DO_PAD), jnp.float32),
    mesh=_sc_mesh(),
    scratch_types=[
        pltpu.VMEM((CHUNK,), jnp.int32),
        pltpu.VMEM((CHUNK,), jnp.int32),
        pltpu.VMEM((CHUNK, DO_PAD), jnp.float32),
        pltpu.VMEM_SHARED((N_PAD, DO_PAD), jnp.float32),
        pltpu.SemaphoreType.DMA,
    ],
)(_agg2_body)


def _mm1_body(x_ref, w_ref, deg_ref, hs_ref, dinv_ref):
    h = jnp.dot(x_ref[...], w_ref[...], preferred_element_type=jnp.float32)
    deg = deg_ref[0, 0] + deg_ref[1, 0] + 1.0
    dinv = lax.rsqrt(deg)
    hs = h * dinv[:, None]
    hs_ref[0] = hs[:, :DHH]
    hs_ref[1] = hs[:, DHH:]
    dinv_ref[0] = dinv


def _mm1(x_p, W1, deg2):
    return pl.pallas_call(
        _mm1_body,
        grid=(GRID,),
        in_specs=[
            pl.BlockSpec((RB, DH), lambda r: (r, 0)),
            pl.BlockSpec((DH, DH), lambda r: (0, 0)),
            pl.BlockSpec((NC, 1, RB), lambda r: (0, r, 0)),
        ],
        out_specs=[
            pl.BlockSpec((NC, RB, DHH), lambda r: (0, r, 0)),
            pl.BlockSpec((1, RB), lambda r: (r, 0)),
        ],
        out_shape=[
            jax.ShapeDtypeStruct((NC, N_PAD, DHH), jnp.float32),
            jax.ShapeDtypeStruct((GRID, RB), jnp.float32),
        ],
    )(x_p, W1, deg2)


def _mm2_body(agg_ref, hs_ref, dinv_ref, b1_ref, w2_ref, zs_ref):
    dinv = dinv_ref[0]
    o0 = jnp.maximum(dinv[:, None] * (agg_ref[0] + hs_ref[0]) + b1_ref[0][None, :], 0.0)
    o1 = jnp.maximum(dinv[:, None] * (agg_ref[1] + hs_ref[1]) + b1_ref[1][None, :], 0.0)
    z = (jnp.dot(o0, w2_ref[0], preferred_element_type=jnp.float32)
         + jnp.dot(o1, w2_ref[1], preferred_element_type=jnp.float32))
    zs_ref[...] = z * dinv[:, None]


def _mm2(agg, hs, dinv, b1s, W2p):
    return pl.pallas_call(
        _mm2_body,
        grid=(GRID,),
        in_specs=[
            pl.BlockSpec((NC, RB, DHH), lambda r: (0, r, 0)),
            pl.BlockSpec((NC, RB, DHH), lambda r: (0, r, 0)),
            pl.BlockSpec((1, RB), lambda r: (r, 0)),
            pl.BlockSpec((NC, DHH), lambda r: (0, 0)),
            pl.BlockSpec((NC, DHH, DO_PAD), lambda r: (0, 0, 0)),
        ],
        out_specs=pl.BlockSpec((RB, DO_PAD), lambda r: (r, 0)),
        out_shape=jax.ShapeDtypeStruct((N_PAD, DO_PAD), jnp.float32),
    )(agg, hs, dinv, b1s, W2p)


def _fin_body(a2_ref, zs_ref, dinv_ref, b2_ref, out_ref):
    dinv = dinv_ref[0]
    t = dinv[:, None] * (a2_ref[0] + a2_ref[1] + zs_ref[...]) + b2_ref[...]
    col = lax.broadcasted_iota(jnp.int32, t.shape, 1)
    valid = col < 7
    neg = jnp.full_like(t, -jnp.inf)
    m = jnp.max(jnp.where(valid, t, neg), axis=1, keepdims=True)
    e = jnp.where(valid, jnp.exp(t - m), 0.0)
    lse = m + jnp.log(jnp.sum(e, axis=1, keepdims=True))
    out_ref[...] = t - lse


def _fin(agg2p, zs, dinv, b2p):
    return pl.pallas_call(
        _fin_body,
        grid=(GRID,),
        in_specs=[
            pl.BlockSpec((NC, RB, DO_PAD), lambda r: (0, r, 0)),
            pl.BlockSpec((RB, DO_PAD), lambda r: (r, 0)),
            pl.BlockSpec((1, RB), lambda r: (r, 0)),
            pl.BlockSpec((1, DO_PAD), lambda r: (0, 0)),
        ],
        out_specs=pl.BlockSpec((RB, DO_PAD), lambda r: (r, 0)),
        out_shape=jax.ShapeDtypeStruct((N_PAD, DO_PAD), jnp.float32),
    )(agg2p, zs, dinv, b2p)


@jax.jit
def kernel(x, edge_index, W1, b1, W2, b2):
    n, _ = x.shape
    e = edge_index.shape[1]
    d_out = W2.shape[1]
    src = edge_index[0].astype(jnp.int32)
    dst = edge_index[1].astype(jnp.int32)
    src_p = jnp.concatenate([src, jnp.zeros((E_PAD - e,), jnp.int32)])
    dst_p = jnp.concatenate([dst, jnp.full((E_PAD - e,), N_PAD - 1, jnp.int32)])
    srcs2 = jnp.stack([src_p, src_p + N_PAD])
    x_p = jnp.pad(x, ((0, N_PAD - n), (0, 0)))
    ones_blk = jnp.ones((CHUNK, DO_PAD), jnp.float32)
    zeros16 = jnp.zeros((ROWS_PER_TILE, DO_PAD), jnp.float32)
    zeros128 = jnp.zeros((ROWS_PER_TILE, DHH), jnp.float32)
    W2p = jnp.pad(W2, ((0, 0), (0, DO_PAD - d_out))).reshape(NC, DHH, DO_PAD)
    b1s = b1.reshape(NC, DHH)
    b2p = jnp.pad(b2, (0, DO_PAD - d_out)).reshape(1, DO_PAD)

    deg_parts = _deg_call(dst_p, ones_blk, zeros16)
    deg2 = deg_parts[:, :, 0].reshape(NC, GRID, RB)
    hs, dinv = _mm1(x_p, W1, deg2)
    agg = _agg1_call(hs.reshape(NC * N_PAD, DHH), srcs2, dst_p, zeros128)
    zs = _mm2(agg, hs, dinv, b1s, W2p)
    agg2p = _agg2_call(zs, src_p, dst_p, zeros16)
    outp = _fin(agg2p, zs, dinv, b2p)
    return outp[:n, :d_out]


# speedup vs baseline: 5.7019x; 5.7019x over previous
import functools

import jax
import jax.numpy as jnp
from jax import lax
from jax.experimental import pallas as pl
from jax.experimental.pallas import tpu as pltpu
from jax.experimental.pallas import tpu_sc as plsc

NC = 2
NS = 16
N_PAD = 10240
ROWS_PER_TILE = N_PAD // NS
E_PAD = 163840
CHUNK = 128
RB = 256
GRID = N_PAD // RB
DH = 256
DHH = DH // NC
DO_PAD = 16


def _sc_mesh():
    return plsc.VectorSubcoreMesh(core_axis_name="c", subcore_axis_name="s")


def _deg_body(dst_hbm, ones_hbm, zeros_hbm, out_hbm, dst_v, ones_v, acc):
    cid = lax.axis_index("c")
    sid = lax.axis_index("s")
    r0 = sid * ROWS_PER_TILE
    pltpu.sync_copy(zeros_hbm, acc.at[pl.ds(r0, ROWS_PER_TILE)])
    pltpu.sync_copy(ones_hbm, ones_v)
    plsc.subcore_barrier()
    e_per_tile = E_PAD // (NC * NS)
    base = cid * (E_PAD // NC) + sid * e_per_tile

    def body(k, carry):
        pltpu.sync_copy(dst_hbm.at[pl.ds(base + k * CHUNK, CHUNK)], dst_v)
        pltpu.sync_copy(ones_v, acc.at[dst_v], add=True)
        return carry

    lax.fori_loop(0, e_per_tile // CHUNK, body, 0)
    plsc.subcore_barrier()
    pltpu.sync_copy(acc.at[pl.ds(r0, ROWS_PER_TILE)],
                    out_hbm.at[cid, pl.ds(r0, ROWS_PER_TILE)])


_deg_call = functools.partial(
    pl.kernel,
    out_type=jax.ShapeDtypeStruct((NC, N_PAD, DO_PAD), jnp.float32),
    mesh=_sc_mesh(),
    scratch_types=[
        pltpu.VMEM((CHUNK,), jnp.int32),
        pltpu.VMEM((CHUNK, DO_PAD), jnp.float32),
        pltpu.VMEM_SHARED((N_PAD, DO_PAD), jnp.float32),
    ],
)(_deg_body)


def _agg1_body(hs_hbm, srcs_hbm, dst_hbm, zeros_hbm, out_hbm,
               src_v, dst_v, rows_v, acc, sem):
    cid = lax.axis_index("c")
    sid = lax.axis_index("s")
    r0 = sid * ROWS_PER_TILE
    pltpu.sync_copy(zeros_hbm, acc.at[pl.ds(r0, ROWS_PER_TILE)])
    plsc.subcore_barrier()
    e_per_tile = E_PAD // NS
    base = sid * e_per_tile

    def body(k, carry):
        b = base + k * CHUNK
        pltpu.sync_copy(srcs_hbm.at[cid, pl.ds(b, CHUNK)], src_v)
        pltpu.sync_copy(dst_hbm.at[pl.ds(b, CHUNK)], dst_v)
        pltpu.async_copy(hs_hbm.at[src_v], rows_v, sem).wait()
        pltpu.sync_copy(rows_v, acc.at[dst_v], add=True)
        return carry

    lax.fori_loop(0, e_per_tile // CHUNK, body, 0)
    plsc.subcore_barrier()
    pltpu.sync_copy(acc.at[pl.ds(r0, ROWS_PER_TILE)],
                    out_hbm.at[cid, pl.ds(r0, ROWS_PER_TILE)])


_agg1_call = functools.partial(
    pl.kernel,
    out_type=jax.ShapeDtypeStruct((NC, N_PAD, DHH), jnp.float32),
    mesh=_sc_mesh(),
    scratch_types=[
        pltpu.VMEM((CHUNK,), jnp.int32),
        pltpu.VMEM((CHUNK,), jnp.int32),
        pltpu.VMEM((CHUNK, DHH), jnp.float32),
        pltpu.VMEM_SHARED((N_PAD, DHH), jnp.float32),
        pltpu.SemaphoreType.DMA,
    ],
)(_agg1_body)


def _agg2_body(zs_hbm, src_hbm, dst_hbm, zeros_hbm, out_hbm,
               src_v, dst_v, rows_v, acc, sem):
    cid = lax.axis_index("c")
    sid = lax.axis_index("s")
    r0 = sid * ROWS_PER_TILE
    pltpu.sync_copy(zeros_hbm, acc.at[pl.ds(r0, ROWS_PER_TILE)])
    plsc.subcore_barrier()
    e_per_tile = E_PAD // (NC * NS)
    base = cid * (E_PAD // NC) + sid * e_per_tile

    def body(k, carry):
        b = base + k * CHUNK
        pltpu.sync_copy(src_hbm.at[pl.ds(b, CHUNK)], src_v)
        pltpu.sync_copy(dst_hbm.at[pl.ds(b, CHUNK)], dst_v)
        pltpu.async_copy(zs_hbm.at[src_v], rows_v, sem).wait()
        pltpu.sync_copy(rows_v, acc.at[dst_v], add=True)
        return carry

    lax.fori_loop(0, e_per_tile // CHUNK, body, 0)
    plsc.subcore_barrier()
    pltpu.sync_copy(acc.at[pl.ds(r0, ROWS_PER_TILE)],
                    out_hbm.at[cid, pl.ds(r0, ROWS_PER_TILE)])


_agg2_call = functools.partial(
    pl.kernel,
    out_type=jax.ShapeDtypeStruct((NC, N_PAD, DHH), jnp.float32),
    mesh=_sc_mesh(),
    scratch_types=[
        pltpu.VMEM((CHUNK,), jnp.int32),
        pltpu.VMEM((CHUNK,), jnp.int32),
        pltpu.VMEM((CHUNK, DHH), jnp.float32),
        pltpu.VMEM_SHARED((N_PAD, DHH), jnp.float32),
        pltpu.SemaphoreType.DMA,
    ],
)(_agg2_body)


def _mm1_body(x_ref, w_ref, deg_ref, hs_ref, dinv_ref):
    h = jnp.dot(x_ref[...], w_ref[...], preferred_element_type=jnp.float32)
    deg = deg_ref[0] + deg_ref[1] + 1.0
    dinv = lax.rsqrt(deg)
    hs = h * dinv[:, None]
    hs_ref[0] = hs[:, :DHH]
    hs_ref[1] = hs[:, DHH:]
    dinv_ref[...] = dinv


def _mm1(x_p, W1, deg2):
    return pl.pallas_call(
        _mm1_body,
        grid=(GRID,),
        in_specs=[
            pl.BlockSpec((RB, DH), lambda r: (r, 0)),
            pl.BlockSpec((DH, DH), lambda r: (0, 0)),
            pl.BlockSpec((NC, RB), lambda r: (0, r)),
        ],
        out_specs=[
            pl.BlockSpec((NC, RB, DHH), lambda r: (0, r, 0)),
            pl.BlockSpec((RB,), lambda r: (r,)),
        ],
        out_shape=[
            jax.ShapeDtypeStruct((NC, N_PAD, DHH), jnp.float32),
            jax.ShapeDtypeStruct((N_PAD,), jnp.float32),
        ],
    )(x_p, W1, deg2)


def _mm2_body(agg_ref, hs_ref, dinv_ref, b1_ref, w2_ref, zs_ref):
    dinv = dinv_ref[...]
    o0 = jnp.maximum(dinv[:, None] * (agg_ref[0] + hs_ref[0]) + b1_ref[0][None, :], 0.0)
    o1 = jnp.maximum(dinv[:, None] * (agg_ref[1] + hs_ref[1]) + b1_ref[1][None, :], 0.0)
    z = (jnp.dot(o0, w2_ref[0], preferred_element_type=jnp.float32)
         + jnp.dot(o1, w2_ref[1], preferred_element_type=jnp.float32))
    zs = z * dinv[:, None]
    zs_ref[...] = jnp.concatenate(
        [zs, jnp.zeros((zs.shape[0], DHH - DO_PAD), jnp.float32)], axis=1)


def _mm2(agg, hs, dinv, b1s, W2p):
    return pl.pallas_call(
        _mm2_body,
        grid=(GRID,),
        in_specs=[
            pl.BlockSpec((NC, RB, DHH), lambda r: (0, r, 0)),
            pl.BlockSpec((NC, RB, DHH), lambda r: (0, r, 0)),
            pl.BlockSpec((RB,), lambda r: (r,)),
            pl.BlockSpec((NC, DHH), lambda r: (0, 0)),
            pl.BlockSpec((NC, DHH, DO_PAD), lambda r: (0, 0, 0)),
        ],
        out_specs=pl.BlockSpec((RB, DHH), lambda r: (r, 0)),
        out_shape=jax.ShapeDtypeStruct((N_PAD, DHH), jnp.float32),
    )(agg, hs, dinv, b1s, W2p)


def _fin_body(a2_ref, zs_ref, dinv_ref, b2_ref, out_ref):
    dinv = dinv_ref[...]
    t = dinv[:, None] * (a2_ref[0] + a2_ref[1] + zs_ref[...]) + b2_ref[...]
    col = lax.broadcasted_iota(jnp.int32, t.shape, 1)
    valid = col < 7
    neg = jnp.full_like(t, -jnp.inf)
    m = jnp.max(jnp.where(valid, t, neg), axis=1, keepdims=True)
    e = jnp.where(valid, jnp.exp(t - m), 0.0)
    lse = m + jnp.log(jnp.sum(e, axis=1, keepdims=True))
    out_ref[...] = t - lse


def _fin(agg2p, zs, dinv, b2p):
    return pl.pallas_call(
        _fin_body,
        grid=(GRID,),
        in_specs=[
            pl.BlockSpec((NC, RB, DHH), lambda r: (0, r, 0)),
            pl.BlockSpec((RB, DHH), lambda r: (r, 0)),
            pl.BlockSpec((RB,), lambda r: (r,)),
            pl.BlockSpec((1, DHH), lambda r: (0, 0)),
        ],
        out_specs=pl.BlockSpec((RB, DHH), lambda r: (r, 0)),
        out_shape=jax.ShapeDtypeStruct((N_PAD, DHH), jnp.float32),
    )(agg2p, zs, dinv, b2p)


@jax.jit
def kernel(x, edge_index, W1, b1, W2, b2):
    n, _ = x.shape
    e = edge_index.shape[1]
    d_out = W2.shape[1]
    src = edge_index[0].astype(jnp.int32)
    dst = edge_index[1].astype(jnp.int32)
    src_p = jnp.concatenate([src, jnp.zeros((E_PAD - e,), jnp.int32)])
    dst_p = jnp.concatenate([dst, jnp.full((E_PAD - e,), N_PAD - 1, jnp.int32)])
    srcs2 = jnp.stack([src_p, src_p + N_PAD])
    x_p = jnp.pad(x, ((0, N_PAD - n), (0, 0)))
    ones_blk = jnp.ones((CHUNK, DO_PAD), jnp.float32)
    zeros16 = jnp.zeros((ROWS_PER_TILE, DO_PAD), jnp.float32)
    zeros128 = jnp.zeros((ROWS_PER_TILE, DHH), jnp.float32)
    W2p = jnp.pad(W2, ((0, 0), (0, DO_PAD - d_out))).reshape(NC, DHH, DO_PAD)
    b1s = b1.reshape(NC, DHH)
    b2p = jnp.pad(b2, (0, DHH - d_out)).reshape(1, DHH)

    deg_parts = _deg_call(dst_p, ones_blk, zeros16)
    deg2 = deg_parts[:, :, 0]
    hs, dinv = _mm1(x_p, W1, deg2)
    agg = _agg1_call(hs.reshape(NC * N_PAD, DHH), srcs2, dst_p, zeros128)
    zs = _mm2(agg, hs, dinv, b1s, W2p)
    agg2p = _agg2_call(zs, src_p, dst_p, zeros128)
    outp = _fin(agg2p, zs, dinv, b2p)
    return outp[:n, :d_out]
